# Initial kernel scaffold; baseline (speedup 1.0000x reference)
import jax
import jax.numpy as jnp
from jax.experimental import pallas as pl


def kernel(embed_idx, ohes, tables, W, b):
    raise NotImplementedError("write your pallas kernel here")


# speedup vs baseline: 1.8853x; 1.8853x over previous
import functools

import jax
import jax.numpy as jnp
from jax import lax
from jax.experimental import pallas as pl
from jax.experimental.pallas import tpu as pltpu
from jax.experimental.pallas import tpu_sc as plsc

N_FIELDS = 26
VOCAB = 100000
EMB = 16
OHE = 100
HID = 128
BATCH = 16384
EMB_FEAT = N_FIELDS * EMB

TOT_ROWS = BATCH * N_FIELDS
NC, NS = 2, 16
NW = NC * NS
ROWS_PER_W = TOT_ROWS // NW
CHUNK = 128
NCHUNK = ROWS_PER_W // CHUNK
NBUF = 4


def _sc_gather_body(idx_hbm, tab_hbm, out_hbm, idx_v, rows_v, *sems):
    wid = lax.axis_index("s") * NC + lax.axis_index("c")
    base_chunk = wid * NCHUNK
    base_row = wid * ROWS_PER_W

    pltpu.sync_copy(idx_hbm.at[pl.ds(base_chunk, NCHUNK)], idx_v)

    def gather_copy(j, b):
        return pltpu.make_async_copy(
            tab_hbm.at[idx_v.at[j]], rows_v.at[b], sems[b])

    for b in range(NBUF):
        gather_copy(b, b).start()

    def step(i, _):
        j0 = i * NBUF
        for b in range(NBUF):
            j = j0 + b
            gather_copy(j, b).wait()
            pltpu.sync_copy(rows_v.at[b],
                            out_hbm.at[pl.ds(base_row + j * CHUNK, CHUNK)])
            nj = j + NBUF

            @pl.when(nj < NCHUNK)
            def _():
                gather_copy(nj, b).start()
        return _

    lax.fori_loop(0, NCHUNK // NBUF, step, None)


_sc_gather = pl.kernel(
    _sc_gather_body,
    out_type=jax.ShapeDtypeStruct((TOT_ROWS, EMB), jnp.float32),
    mesh=plsc.VectorSubcoreMesh(core_axis_name="c", subcore_axis_name="s"),
    compiler_params=pltpu.CompilerParams(use_tc_tiling_on_sc=False),
    scratch_types=(
        [pltpu.VMEM((NCHUNK, CHUNK), jnp.int32),
         pltpu.VMEM((NBUF, CHUNK, EMB), jnp.float32)]
        + [pltpu.SemaphoreType.DMA] * NBUF
    ),
)


def _mm_body(g_ref, o_ref, w1_ref, w2_ref, b_ref, out_ref):
    acc = jnp.dot(g_ref[...], w1_ref[...], preferred_element_type=jnp.float32)
    acc += jnp.dot(o_ref[...], w2_ref[...], preferred_element_type=jnp.float32)
    out_ref[...] = acc + b_ref[...]


def _dense(g2, ohes, w1, w2, b2):
    bm = 1024
    return pl.pallas_call(
        _mm_body,
        grid=(BATCH // bm,),
        in_specs=[
            pl.BlockSpec((bm, EMB_FEAT), lambda m: (m, 0)),
            pl.BlockSpec((bm, OHE), lambda m: (m, 0)),
            pl.BlockSpec((EMB_FEAT, HID), lambda m: (0, 0)),
            pl.BlockSpec((OHE, HID), lambda m: (0, 0)),
            pl.BlockSpec((1, HID), lambda m: (0, 0)),
        ],
        out_specs=pl.BlockSpec((bm, HID), lambda m: (m, 0)),
        out_shape=jax.ShapeDtypeStruct((BATCH, HID), jnp.float32),
    )(g2, ohes, w1, w2, b2)


def kernel(embed_idx, ohes, tables, W, b):
    offs = (jnp.arange(N_FIELDS, dtype=jnp.int32) * VOCAB)[None, :]
    idx2d = (embed_idx.astype(jnp.int32) + offs).reshape(TOT_ROWS // CHUNK,
                                                         CHUNK)
    tab_flat = tables.reshape(N_FIELDS * VOCAB, EMB)
    g = _sc_gather(idx2d, tab_flat)
    g2 = g.reshape(BATCH, EMB_FEAT)
    return _dense(g2, ohes, W[:EMB_FEAT], W[EMB_FEAT:], b.reshape(1, HID))
